# baseline (speedup 1.0000x reference)
import jax
import jax.numpy as jnp
from jax.experimental import pallas as pl
from jax.experimental.pallas import tpu as pltpu

_EPS = 1e-12


def _main_kernel(q_ref, cm2_ref, m2p1_ref, dist_ref, pred_ref):
    q = q_ref[...]
    n = jnp.sqrt(jnp.sum(q * q, axis=1, keepdims=True))
    qn = q / jnp.maximum(n, _EPS)
    cm2 = cm2_ref[...]
    m2p1 = m2p1_ref[...]
    prod2 = jax.lax.dot_general(cm2, qn, (((1,), (1,)), ((), ())),
                                preferred_element_type=jnp.float32)
    sq = jnp.maximum(m2p1 - prod2, 0.0)
    y = sq + 1e-12
    dist_ref[...] = y * jax.lax.rsqrt(y)
    pred_ref[...] = jnp.argmin(sq, axis=0).astype(jnp.int32)[None, :]


def kernel(queries, exemplar_feats):
    Q, D = queries.shape
    C, M, D2 = exemplar_feats.shape

    n_ex = jnp.sqrt(jnp.sum(exemplar_feats * exemplar_feats, axis=-1,
                            keepdims=True))
    exn = exemplar_feats / jnp.maximum(n_ex, _EPS)
    cm = jnp.mean(exn, axis=1)
    n_cm = jnp.sqrt(jnp.sum(cm * cm, axis=-1, keepdims=True))
    means = cm / jnp.maximum(n_cm, _EPS)
    m2p1 = 1.0 + jnp.sum(means * means, axis=1)[:, None]
    means2 = means + means

    BQ = 1024
    dists_t, preds = pl.pallas_call(
        _main_kernel,
        grid=(Q // BQ,),
        in_specs=[
            pl.BlockSpec((BQ, D), lambda i: (i, 0)),
            pl.BlockSpec((C, D2), lambda i: (0, 0)),
            pl.BlockSpec((C, 1), lambda i: (0, 0)),
        ],
        out_specs=[
            pl.BlockSpec((C, BQ), lambda i: (0, i)),
            pl.BlockSpec((1, BQ), lambda i: (0, i)),
        ],
        out_shape=[
            jax.ShapeDtypeStruct((C, Q), jnp.float32),
            jax.ShapeDtypeStruct((1, Q), jnp.int32),
        ],
        compiler_params=pltpu.CompilerParams(
            dimension_semantics=("parallel",)),
    )(queries, means2, m2p1)

    return dists_t.T, preds[0]


# speedup vs baseline: 1.1246x; 1.1246x over previous
import jax
import jax.numpy as jnp
from jax.experimental import pallas as pl
from jax.experimental.pallas import tpu as pltpu

_EPS = 1e-12


def _main_kernel(q_ref, cm2_ref, m2p1_ref, dist_ref, pred_ref):
    q = q_ref[...]
    n = jnp.sqrt(jnp.sum(q * q, axis=1, keepdims=True))
    qn = q / jnp.maximum(n, _EPS)
    cm2 = cm2_ref[...]
    m2p1 = m2p1_ref[...]
    prod2 = jax.lax.dot_general(cm2, qn, (((1,), (1,)), ((), ())),
                                preferred_element_type=jnp.float32)
    sq = jnp.maximum(m2p1 - prod2, 0.0)
    y = sq + 1e-12
    dist_ref[...] = y * jax.lax.rsqrt(y)
    C = sq.shape[0]
    val = sq[0:8, :]
    idx = jnp.zeros(val.shape, jnp.int32)
    for r in range(1, C // 8):
        chunk = jax.lax.slice_in_dim(sq, 8 * r, 8 * r + 8, axis=0)
        lt = chunk < val
        val = jnp.where(lt, chunk, val)
        idx = jnp.where(lt, jnp.int32(r), idx)
    m8 = jnp.min(val, axis=0, keepdims=True)
    cls = idx * 8 + jax.lax.broadcasted_iota(jnp.int32, val.shape, 0)
    cand = jnp.where(val == m8, cls, jnp.int32(2 ** 30))
    pred_ref[...] = jnp.min(cand, axis=0)[None, :]


def kernel(queries, exemplar_feats):
    Q, D = queries.shape
    C, M, D2 = exemplar_feats.shape

    n_ex = jnp.sqrt(jnp.sum(exemplar_feats * exemplar_feats, axis=-1,
                            keepdims=True))
    exn = exemplar_feats / jnp.maximum(n_ex, _EPS)
    cm = jnp.mean(exn, axis=1)
    n_cm = jnp.sqrt(jnp.sum(cm * cm, axis=-1, keepdims=True))
    means = cm / jnp.maximum(n_cm, _EPS)
    m2p1 = 1.0 + jnp.sum(means * means, axis=1)[:, None]
    means2 = means + means

    BQ = 2048
    dists_t, preds = pl.pallas_call(
        _main_kernel,
        grid=(Q // BQ,),
        in_specs=[
            pl.BlockSpec((BQ, D), lambda i: (i, 0)),
            pl.BlockSpec((C, D2), lambda i: (0, 0)),
            pl.BlockSpec((C, 1), lambda i: (0, 0)),
        ],
        out_specs=[
            pl.BlockSpec((C, BQ), lambda i: (0, i)),
            pl.BlockSpec((1, BQ), lambda i: (0, i)),
        ],
        out_shape=[
            jax.ShapeDtypeStruct((C, Q), jnp.float32),
            jax.ShapeDtypeStruct((1, Q), jnp.int32),
        ],
        compiler_params=pltpu.CompilerParams(
            dimension_semantics=("parallel",)),
    )(queries, means2, m2p1)

    return dists_t.T, preds[0]
